# Initial kernel scaffold; baseline (speedup 1.0000x reference)
import jax
import jax.numpy as jnp
from jax.experimental import pallas as pl


def kernel(x, edge_index, W1, b1, W2, b2):
    raise NotImplementedError("write your pallas kernel here")


# speedup vs baseline: 13.9517x; 13.9517x over previous
import functools

import jax
import jax.numpy as jnp
from jax import lax
from jax.experimental import pallas as pl
from jax.experimental.pallas import tpu as pltpu
from jax.experimental.pallas import tpu_sc as plsc

_N = 10000
_E = 160000
_DIN = 256
_DH = 128
_DOUT = 256
_NP = 10240
_HALF = _DH // 2
_CH = 128
_NCK = _NP // _CH
_RB = 1024
_GRID = _NP // _RB

_sc_mesh = plsc.VectorSubcoreMesh(core_axis_name="c", subcore_axis_name="s")


def _deg_body(dst_hbm, out_hbm, dstv, hist):
    c = lax.axis_index("c")
    s = lax.axis_index("s")
    half = _NCK // 2
    pltpu.sync_copy(dst_hbm.at[s, pl.ds(c * half, half)], dstv)

    def zb(i, carry):
        hist[pl.ds(i * 16, 16)] = jnp.zeros((16,), jnp.float32)
        return carry

    lax.fori_loop(0, _NP // 16, zb, 0)
    ones = jnp.full((16,), 1.0, jnp.float32)

    def hb(i, carry):
        for k in range(_CH // 16):
            idx = dstv[i, pl.ds(k * 16, 16)]
            plsc.addupdate_scatter(hist, [idx], ones)
        return carry

    lax.fori_loop(0, half, hb, 0)
    wid = c * 16 + s
    pltpu.sync_copy(hist, out_hbm.at[wid])


_deg_kernel = functools.partial(
    pl.kernel,
    out_type=jax.ShapeDtypeStruct((32, _NP), jnp.float32),
    mesh=_sc_mesh,
    compiler_params=pltpu.CompilerParams(needs_layout_passes=False),
    scratch_types=[
        pltpu.VMEM((_NCK // 2, _CH), jnp.int32),
        pltpu.VMEM((_NP,), jnp.float32),
    ],
)(_deg_body)


def _scat_body(z_hbm, src_hbm, dst_hbm, out_hbm, srcv, dstv, g0, g1, acc, s0, s1):
    c = lax.axis_index("c")
    s = lax.axis_index("s")
    pltpu.sync_copy(src_hbm.at[c, s], srcv)
    pltpu.sync_copy(dst_hbm.at[s], dstv)

    def zb(i, carry):
        for k in range(4):
            g0[i, pl.ds(k * 16, 16)] = jnp.zeros((16,), jnp.float32)
        return carry

    lax.fori_loop(0, _CH, zb, 0)
    rows_per_tile = _NP // 16
    for k in range(rows_per_tile // _CH):
        pltpu.sync_copy(g0, acc.at[pl.ds(s * rows_per_tile + k * _CH, _CH)])
    plsc.subcore_barrier()

    def start(k, buf, sem):
        pltpu.async_copy(z_hbm.at[srcv.at[k]], buf, sem)

    def wait(buf, sem):
        pltpu.make_async_copy(z_hbm.at[pl.ds(0, _CH)], buf, sem).wait()

    def scat(k, buf):
        pltpu.sync_copy(buf, acc.at[dstv.at[k]], add=True)

    start(0, g0, s0)
    start(1, g1, s1)

    def body(j, carry):
        k = j * 2
        wait(g0, s0)
        scat(k, g0)

        @pl.when(k + 2 < _NCK)
        def _():
            start(k + 2, g0, s0)

        wait(g1, s1)
        scat(k + 1, g1)

        @pl.when(k + 3 < _NCK)
        def _():
            start(k + 3, g1, s1)

        return carry

    lax.fori_loop(0, _NCK // 2, body, 0)
    plsc.subcore_barrier()

    def wb(k, carry):
        base = s * rows_per_tile + k * _CH
        pltpu.sync_copy(acc.at[pl.ds(base, _CH)], g0)
        pltpu.sync_copy(g0, out_hbm.at[c, pl.ds(base, _CH)])
        return carry

    lax.fori_loop(0, rows_per_tile // _CH, wb, 0)


_scat_kernel = functools.partial(
    pl.kernel,
    out_type=jax.ShapeDtypeStruct((2, _NP, _HALF), jnp.float32),
    mesh=_sc_mesh,
    compiler_params=pltpu.CompilerParams(use_tc_tiling_on_sc=False),
    scratch_types=[
        pltpu.VMEM((_NCK, _CH), jnp.int32),
        pltpu.VMEM((_NCK, _CH), jnp.int32),
        pltpu.VMEM((_CH, _HALF), jnp.float32),
        pltpu.VMEM((_CH, _HALF), jnp.float32),
        pltpu.VMEM_SHARED((_NP, _HALF), jnp.float32),
        pltpu.SemaphoreType.DMA,
        pltpu.SemaphoreType.DMA,
    ],
)(_scat_body)


def _mm1_body(x_ref, w_ref, cnt_ref, z_ref, dinv_ref):
    i = pl.program_id(0)
    rows = i * _RB + lax.broadcasted_iota(jnp.int32, (_RB, 1), 0)
    valid = rows < _N
    xb = jnp.where(valid, x_ref[...], 0.0)
    h = jnp.dot(xb, w_ref[...], preferred_element_type=jnp.float32)
    deg = 1.0 + jnp.sum(cnt_ref[...], axis=0)[:, None]
    dinv = lax.rsqrt(deg)
    z = jnp.where(valid, h * dinv, 0.0)
    z_ref[0] = z[:, :_HALF]
    z_ref[1] = z[:, _HALF:]
    dinv_ref[...] = dinv


def _mm1(x, W1, cnt):
    return pl.pallas_call(
        _mm1_body,
        grid=(_GRID,),
        in_specs=[
            pl.BlockSpec((_RB, _DIN), lambda i: (i, 0)),
            pl.BlockSpec((_DIN, _DH), lambda i: (0, 0)),
            pl.BlockSpec((32, _RB), lambda i: (0, i)),
        ],
        out_specs=[
            pl.BlockSpec((2, _RB, _HALF), lambda i: (0, i, 0)),
            pl.BlockSpec((_RB, 1), lambda i: (i, 0)),
        ],
        out_shape=[
            jax.ShapeDtypeStruct((2, _NP, _HALF), jnp.float32),
            jax.ShapeDtypeStruct((_NP, 1), jnp.float32),
        ],
    )(x, W1, cnt)


def _mid_body(z1_ref, s1_ref, dinv_ref, b1_ref, z2_ref):
    i = pl.program_id(0)
    rows = i * _RB + lax.broadcasted_iota(jnp.int32, (_RB, 1), 0)
    valid = rows < _N
    dinv = dinv_ref[...]
    for c in range(2):
        t = dinv * (z1_ref[c] + s1_ref[c]) + b1_ref[c][None, :]
        z2_ref[c] = jnp.where(valid, dinv * jnp.maximum(t, 0.0), 0.0)


def _mid(z1, s1, dinv, b1):
    return pl.pallas_call(
        _mid_body,
        grid=(_GRID,),
        in_specs=[
            pl.BlockSpec((2, _RB, _HALF), lambda i: (0, i, 0)),
            pl.BlockSpec((2, _RB, _HALF), lambda i: (0, i, 0)),
            pl.BlockSpec((_RB, 1), lambda i: (i, 0)),
            pl.BlockSpec((2, _HALF), lambda i: (0, 0)),
        ],
        out_specs=pl.BlockSpec((2, _RB, _HALF), lambda i: (0, i, 0)),
        out_shape=jax.ShapeDtypeStruct((2, _NP, _HALF), jnp.float32),
    )(z1, s1, dinv, b1)


def _mm2_body(z2_ref, s2_ref, dinv_ref, w_ref, b_ref, o_ref):
    dinv = dinv_ref[...]
    a0 = dinv * (z2_ref[0] + s2_ref[0])
    a1 = dinv * (z2_ref[1] + s2_ref[1])
    agg = jnp.concatenate([a0, a1], axis=1)
    o_ref[...] = jnp.dot(agg, w_ref[...], preferred_element_type=jnp.float32) + b_ref[...]


def _mm2(z2, s2, dinv, W2, b2):
    return pl.pallas_call(
        _mm2_body,
        grid=(_GRID,),
        in_specs=[
            pl.BlockSpec((2, _RB, _HALF), lambda i: (0, i, 0)),
            pl.BlockSpec((2, _RB, _HALF), lambda i: (0, i, 0)),
            pl.BlockSpec((_RB, 1), lambda i: (i, 0)),
            pl.BlockSpec((_DH, _DOUT), lambda i: (0, 0)),
            pl.BlockSpec((1, _DOUT), lambda i: (0, 0)),
        ],
        out_specs=pl.BlockSpec((_RB, _DOUT), lambda i: (i, 0)),
        out_shape=jax.ShapeDtypeStruct((_N, _DOUT), jnp.float32),
    )(z2, s2, dinv, W2, b2)


def kernel(x, edge_index, W1, b1, W2, b2):
    src = edge_index[0].astype(jnp.int32)
    dst = edge_index[1].astype(jnp.int32)
    per_tile = _E // 16
    pad = jnp.full((16, _NP - per_tile), _N, jnp.int32)
    srcr = jnp.concatenate([src.reshape(16, per_tile), pad], axis=1)
    srcr = srcr.reshape(16, _NCK, _CH)
    dstr = jnp.concatenate([dst.reshape(16, per_tile), pad], axis=1)
    dstr = dstr.reshape(16, _NCK, _CH)
    src2 = jnp.stack([srcr, srcr + _NP])

    cnt = _deg_kernel(dstr)
    z1, dinv = _mm1(x, W1, cnt)
    s1 = _scat_kernel(z1.reshape(2 * _NP, _HALF), src2, dstr)
    z2 = _mid(z1, s1, dinv, b1.reshape(2, _HALF))
    s2 = _scat_kernel(z2.reshape(2 * _NP, _HALF), src2, dstr)
    return _mm2(z2, s2, dinv, W2, b2.reshape(1, _DOUT))


# speedup vs baseline: 14.2526x; 1.0216x over previous
import functools

import jax
import jax.numpy as jnp
from jax import lax
from jax.experimental import pallas as pl
from jax.experimental.pallas import tpu as pltpu
from jax.experimental.pallas import tpu_sc as plsc

_N = 10000
_E = 160000
_DIN = 256
_DH = 128
_DOUT = 256
_NP = 10240
_HALF = _DH // 2
_CH = 128
_NCK = _NP // _CH
_RB = 1024
_GRID = _NP // _RB

_sc_mesh = plsc.VectorSubcoreMesh(core_axis_name="c", subcore_axis_name="s")


def _deg_body(dst_hbm, out_hbm, dstv, hist):
    c = lax.axis_index("c")
    s = lax.axis_index("s")
    half = _NCK // 2
    pltpu.sync_copy(dst_hbm.at[s, pl.ds(c * half, half)], dstv)

    def zb(i, carry):
        hist[pl.ds(i * 16, 16)] = jnp.zeros((16,), jnp.float32)
        return carry

    lax.fori_loop(0, _NP // 16, zb, 0)
    ones = jnp.full((16,), 1.0, jnp.float32)

    def hb(i, carry):
        for k in range(_CH // 16):
            idx = dstv[i, pl.ds(k * 16, 16)]
            plsc.addupdate_scatter(hist, [idx], ones)
        return carry

    lax.fori_loop(0, half, hb, 0)
    wid = c * 16 + s
    pltpu.sync_copy(hist, out_hbm.at[wid])


_deg_kernel = functools.partial(
    pl.kernel,
    out_type=jax.ShapeDtypeStruct((32, _NP), jnp.float32),
    mesh=_sc_mesh,
    compiler_params=pltpu.CompilerParams(needs_layout_passes=False),
    scratch_types=[
        pltpu.VMEM((_NCK // 2, _CH), jnp.int32),
        pltpu.VMEM((_NP,), jnp.float32),
    ],
)(_deg_body)


_KG = 4
_NG = _NCK // (2 * _KG)


def _scat_body(z_hbm, src_hbm, dst_hbm, out_hbm, srcv, dstv, bufs, acc,
               sgA, sgB, ssA, ssB):
    c = lax.axis_index("c")
    s = lax.axis_index("s")
    pltpu.sync_copy(src_hbm.at[c, s], srcv)
    pltpu.sync_copy(dst_hbm.at[s], dstv)

    def zb(i, carry):
        for b in range(_KG):
            for k in range(4):
                bufs[b, i, pl.ds(k * 16, 16)] = jnp.zeros((16,), jnp.float32)
        return carry

    lax.fori_loop(0, _CH, zb, 0)
    rows_per_tile = _NP // 16
    for k in range(rows_per_tile // _CH):
        pltpu.sync_copy(bufs.at[0], acc.at[pl.ds(s * rows_per_tile + k * _CH, _CH)])
    plsc.subcore_barrier()

    def fire_gathers(g, base_b, sem):
        for b in range(_KG):
            pltpu.async_copy(z_hbm.at[srcv.at[g * _KG + b]], bufs.at[base_b + b], sem)

    def drain_gathers(base_b, sem):
        for b in range(_KG):
            pltpu.make_async_copy(z_hbm.at[pl.ds(0, _CH)], bufs.at[base_b + b], sem).wait()

    def fire_scatters(g, base_b, sem):
        for b in range(_KG):
            pltpu.async_copy(bufs.at[base_b + b], acc.at[dstv.at[g * _KG + b]], sem,
                             add=True)

    def drain_scatters(base_b, sem):
        for b in range(_KG):
            pltpu.make_async_copy(z_hbm.at[pl.ds(0, _CH)],
                                  acc.at[pl.ds(0, _CH)], sem).wait()

    fire_gathers(0, 0, sgA)
    fire_gathers(1, _KG, sgB)

    def body(gi, carry):
        g = gi * 2
        drain_gathers(0, sgA)
        fire_scatters(g, 0, ssA)
        drain_scatters(0, ssA)

        @pl.when(g + 2 < 2 * _NG)
        def _():
            fire_gathers(g + 2, 0, sgA)

        drain_gathers(_KG, sgB)
        fire_scatters(g + 1, _KG, ssB)
        drain_scatters(_KG, ssB)

        @pl.when(g + 3 < 2 * _NG)
        def _():
            fire_gathers(g + 3, _KG, sgB)

        return carry

    lax.fori_loop(0, _NG, body, 0)
    plsc.subcore_barrier()

    def wb(k, carry):
        base = s * rows_per_tile + k * _CH
        pltpu.sync_copy(acc.at[pl.ds(base, _CH)], out_hbm.at[c, pl.ds(base, _CH)])
        return carry

    lax.fori_loop(0, rows_per_tile // _CH, wb, 0)


_scat_kernel = functools.partial(
    pl.kernel,
    out_type=jax.ShapeDtypeStruct((2, _NP, _HALF), jnp.float32),
    mesh=_sc_mesh,
    compiler_params=pltpu.CompilerParams(use_tc_tiling_on_sc=False),
    scratch_types=[
        pltpu.VMEM((_NCK, _CH), jnp.int32),
        pltpu.VMEM((_NCK, _CH), jnp.int32),
        pltpu.VMEM((2 * _KG, _CH, _HALF), jnp.float32),
        pltpu.VMEM_SHARED((_NP, _HALF), jnp.float32),
        pltpu.SemaphoreType.DMA,
        pltpu.SemaphoreType.DMA,
        pltpu.SemaphoreType.DMA,
        pltpu.SemaphoreType.DMA,
    ],
)(_scat_body)


def _mm1_body(x_ref, w_ref, cnt_ref, z_ref, dinv_ref):
    i = pl.program_id(0)
    rows = i * _RB + lax.broadcasted_iota(jnp.int32, (_RB, 1), 0)
    valid = rows < _N
    xb = jnp.where(valid, x_ref[...], 0.0)
    h = jnp.dot(xb, w_ref[...], preferred_element_type=jnp.float32)
    deg = 1.0 + jnp.sum(cnt_ref[...], axis=0)[:, None]
    dinv = lax.rsqrt(deg)
    z = jnp.where(valid, h * dinv, 0.0)
    z_ref[0] = z[:, :_HALF]
    z_ref[1] = z[:, _HALF:]
    dinv_ref[...] = dinv


def _mm1(x, W1, cnt):
    return pl.pallas_call(
        _mm1_body,
        grid=(_GRID,),
        in_specs=[
            pl.BlockSpec((_RB, _DIN), lambda i: (i, 0)),
            pl.BlockSpec((_DIN, _DH), lambda i: (0, 0)),
            pl.BlockSpec((32, _RB), lambda i: (0, i)),
        ],
        out_specs=[
            pl.BlockSpec((2, _RB, _HALF), lambda i: (0, i, 0)),
            pl.BlockSpec((_RB, 1), lambda i: (i, 0)),
        ],
        out_shape=[
            jax.ShapeDtypeStruct((2, _NP, _HALF), jnp.float32),
            jax.ShapeDtypeStruct((_NP, 1), jnp.float32),
        ],
    )(x, W1, cnt)


def _mid_body(z1_ref, s1_ref, dinv_ref, b1_ref, z2_ref):
    i = pl.program_id(0)
    rows = i * _RB + lax.broadcasted_iota(jnp.int32, (_RB, 1), 0)
    valid = rows < _N
    dinv = dinv_ref[...]
    for c in range(2):
        t = dinv * (z1_ref[c] + s1_ref[c]) + b1_ref[c][None, :]
        z2_ref[c] = jnp.where(valid, dinv * jnp.maximum(t, 0.0), 0.0)


def _mid(z1, s1, dinv, b1):
    return pl.pallas_call(
        _mid_body,
        grid=(_GRID,),
        in_specs=[
            pl.BlockSpec((2, _RB, _HALF), lambda i: (0, i, 0)),
            pl.BlockSpec((2, _RB, _HALF), lambda i: (0, i, 0)),
            pl.BlockSpec((_RB, 1), lambda i: (i, 0)),
            pl.BlockSpec((2, _HALF), lambda i: (0, 0)),
        ],
        out_specs=pl.BlockSpec((2, _RB, _HALF), lambda i: (0, i, 0)),
        out_shape=jax.ShapeDtypeStruct((2, _NP, _HALF), jnp.float32),
    )(z1, s1, dinv, b1)


def _mm2_body(z2_ref, s2_ref, dinv_ref, w_ref, b_ref, o_ref):
    dinv = dinv_ref[...]
    a0 = dinv * (z2_ref[0] + s2_ref[0])
    a1 = dinv * (z2_ref[1] + s2_ref[1])
    agg = jnp.concatenate([a0, a1], axis=1)
    o_ref[...] = jnp.dot(agg, w_ref[...], preferred_element_type=jnp.float32) + b_ref[...]


def _mm2(z2, s2, dinv, W2, b2):
    return pl.pallas_call(
        _mm2_body,
        grid=(_GRID,),
        in_specs=[
            pl.BlockSpec((2, _RB, _HALF), lambda i: (0, i, 0)),
            pl.BlockSpec((2, _RB, _HALF), lambda i: (0, i, 0)),
            pl.BlockSpec((_RB, 1), lambda i: (i, 0)),
            pl.BlockSpec((_DH, _DOUT), lambda i: (0, 0)),
            pl.BlockSpec((1, _DOUT), lambda i: (0, 0)),
        ],
        out_specs=pl.BlockSpec((_RB, _DOUT), lambda i: (i, 0)),
        out_shape=jax.ShapeDtypeStruct((_N, _DOUT), jnp.float32),
    )(z2, s2, dinv, W2, b2)


def kernel(x, edge_index, W1, b1, W2, b2):
    src = edge_index[0].astype(jnp.int32)
    dst = edge_index[1].astype(jnp.int32)
    per_tile = _E // 16
    pad = jnp.full((16, _NP - per_tile), _N, jnp.int32)
    srcr = jnp.concatenate([src.reshape(16, per_tile), pad], axis=1)
    srcr = srcr.reshape(16, _NCK, _CH)
    dstr = jnp.concatenate([dst.reshape(16, per_tile), pad], axis=1)
    dstr = dstr.reshape(16, _NCK, _CH)
    src2 = jnp.stack([srcr, srcr + _NP])

    cnt = _deg_kernel(dstr)
    z1, dinv = _mm1(x, W1, cnt)
    s1 = _scat_kernel(z1.reshape(2 * _NP, _HALF), src2, dstr)
    z2 = _mid(z1, s1, dinv, b1.reshape(2, _HALF))
    s2 = _scat_kernel(z2.reshape(2 * _NP, _HALF), src2, dstr)
    return _mm2(z2, s2, dinv, W2, b2.reshape(1, _DOUT))
